# Initial kernel scaffold; baseline (speedup 1.0000x reference)
import jax
import jax.numpy as jnp
from jax.experimental import pallas as pl


def kernel(x):
    raise NotImplementedError("write your pallas kernel here")


# speedup vs baseline: 3.6122x; 3.6122x over previous
import functools

import jax
import jax.numpy as jnp
from jax.experimental import pallas as pl
from jax.experimental.pallas import tpu as pltpu

N = 1024
T = 32
BR = 256


def _dummies_body(x_ref, out_ref):
    r = pl.program_id(1)
    xv = x_ref[0, 0, :]
    mask = jnp.where(jnp.isnan(xv), 0.0, 1.0)
    ii = jax.lax.broadcasted_iota(jnp.int32, (N, N), 0)
    jj = jax.lax.broadcasted_iota(jnp.int32, (N, N), 1)
    tri = jnp.where(ii <= jj, 1.0, 0.0)
    dest = jax.lax.dot_general(
        mask.reshape(1, N), tri,
        (((1,), (0,)), ((), ())),
        preferred_element_type=jnp.float32,
    ) - 1.0
    dest1 = dest[:, 1:]
    mask1 = mask.reshape(1, N)[:, 1:]
    rows = (jax.lax.broadcasted_iota(jnp.int32, (BR, 1), 0)
            + r * BR).astype(jnp.float32)
    out_ref[...] = jnp.where((dest1 == rows) & (mask1 > 0.0), 1.0, 0.0)


@jax.jit
def kernel(x):
    xs = x.reshape(T, 1, N)
    out = pl.pallas_call(
        _dummies_body,
        grid=(T, N // BR),
        in_specs=[pl.BlockSpec((1, 1, N), lambda t, r: (t, 0, 0))],
        out_specs=pl.BlockSpec(
            (BR, N - 1), lambda t, r: (t * (N // BR) + r, 0)
        ),
        out_shape=jax.ShapeDtypeStruct((T * N, N - 1), jnp.float32),
    )(xs)
    return out.reshape(1, T * N, N - 1)


# speedup vs baseline: 4.6431x; 1.2854x over previous
import jax
import jax.numpy as jnp
from jax.experimental import pallas as pl
from jax.experimental.pallas import tpu as pltpu

N = 1024
T = 32
BR = 512
R = N // BR


def _dummies_body(x_ref, out_ref, tgt_ref):
    t = pl.program_id(0)
    r = pl.program_id(1)

    @pl.when((t == 0) & (r == 0))
    def _build_targets():
        xm = x_ref[:, 0, :]
        mask = jnp.where(jnp.isnan(xm), 0.0, 1.0)
        ii = jax.lax.broadcasted_iota(jnp.int32, (N, N), 0)
        jj = jax.lax.broadcasted_iota(jnp.int32, (N, N), 1)
        tri = jnp.where(ii <= jj, 1.0, 0.0)
        dest = jax.lax.dot_general(
            mask, tri, (((1,), (0,)), ((), ())),
            preferred_element_type=jnp.float32,
        ) - 1.0
        tgt_ref[...] = jnp.where(mask[:, 1:] > 0.0, dest[:, 1:], -2.0)

    trow = tgt_ref[pl.ds(t, 1), :]
    rows = (jax.lax.broadcasted_iota(jnp.int32, (BR, 1), 0)
            + r * BR).astype(jnp.float32)
    out_ref[...] = jnp.where(trow == rows, 1.0, 0.0)


@jax.jit
def kernel(x):
    xs = x.reshape(T, 1, N)
    out = pl.pallas_call(
        _dummies_body,
        grid=(T, R),
        in_specs=[pl.BlockSpec((T, 1, N), lambda t, r: (0, 0, 0))],
        out_specs=pl.BlockSpec((BR, N - 1), lambda t, r: (t * R + r, 0)),
        out_shape=jax.ShapeDtypeStruct((T * N, N - 1), jnp.float32),
        scratch_shapes=[pltpu.VMEM((T, N - 1), jnp.float32)],
    )(xs)
    return out.reshape(1, T * N, N - 1)


# speedup vs baseline: 4.8350x; 1.0413x over previous
import jax
import jax.numpy as jnp
from jax.experimental import pallas as pl
from jax.experimental.pallas import tpu as pltpu

N = 1024
T = 32
BR = 1024
R = N // BR


def _dummies_body(x_ref, out_ref, tgt_ref):
    t = pl.program_id(0)
    r = pl.program_id(1)

    @pl.when((t == 0) & (r == 0))
    def _build_targets():
        xm = x_ref[:, 0, :]
        mask = jnp.where(jnp.isnan(xm), 0.0, 1.0)
        ii = jax.lax.broadcasted_iota(jnp.int32, (N, N), 0)
        jj = jax.lax.broadcasted_iota(jnp.int32, (N, N), 1)
        tri = jnp.where(ii <= jj, 1.0, 0.0)
        dest = jax.lax.dot_general(
            mask, tri, (((1,), (0,)), ((), ())),
            preferred_element_type=jnp.float32,
        ) - 1.0
        tgt_ref[...] = jnp.where(mask[:, 1:] > 0.0, dest[:, 1:], -2.0)

    trow = tgt_ref[pl.ds(t, 1), :]
    rows = (jax.lax.broadcasted_iota(jnp.int32, (BR, 1), 0)
            + r * BR).astype(jnp.float32)
    out_ref[...] = jnp.where(trow == rows, 1.0, 0.0)


@jax.jit
def kernel(x):
    xs = x.reshape(T, 1, N)
    out = pl.pallas_call(
        _dummies_body,
        grid=(T, R),
        in_specs=[pl.BlockSpec((T, 1, N), lambda t, r: (0, 0, 0))],
        out_specs=pl.BlockSpec((BR, N - 1), lambda t, r: (t * R + r, 0)),
        out_shape=jax.ShapeDtypeStruct((T * N, N - 1), jnp.float32),
        scratch_shapes=[pltpu.VMEM((T, N - 1), jnp.float32)],
    )(xs)
    return out.reshape(1, T * N, N - 1)
